# baseline (speedup 1.0000x reference)
import functools

import jax
import jax.numpy as jnp
from jax import lax
from jax.experimental import pallas as pl
from jax.experimental.pallas import tpu as pltpu
from jax.experimental.pallas import tpu_sc as plsc

_NC = 2
_NS = 16
_NW = _NC * _NS
_L = 16
_CH = 256


@functools.lru_cache(maxsize=None)
def _make_gather(B: int, D: int):
    assert B % (8 * _NW) == 0
    bpw = B // _NW
    D2 = 2 * D

    mesh = plsc.VectorSubcoreMesh(core_axis_name="c", subcore_axis_name="s")

    @functools.partial(
        pl.kernel,
        out_type=jax.ShapeDtypeStruct((B, D), jnp.float32),
        mesh=mesh,
        scratch_types=[
            pltpu.VMEM((bpw,), jnp.int32),
            pltpu.VMEM((bpw,), jnp.int32),
            pltpu.VMEM((_CH, D2), jnp.float32),
            pltpu.VMEM((bpw, D), jnp.float32),
            pltpu.SemaphoreType.DMA,
        ],
        compiler_params=pltpu.CompilerParams(
            use_tc_tiling_on_sc=False, needs_layout_passes=False),
    )
    def gather(table2_hbm, idx_hbm, out_hbm, ids_v, rows_i, buf_v, out_v, sem):
        wid = lax.axis_index("s") * _NC + lax.axis_index("c")
        base = wid * bpw
        pltpu.sync_copy(idx_hbm.at[pl.ds(base, bpw)], ids_v)

        def mk_rows(k, _):
            v = ids_v[pl.ds(k * _L, _L)]
            rows_i[pl.ds(k * _L, _L)] = jax.lax.shift_right_logical(v, 1)
            return _
        lax.fori_loop(0, bpw // _L, mk_rows, 0, unroll=4)

        for c in range(bpw // _CH):
            pltpu.async_copy(
                table2_hbm.at[rows_i.at[pl.ds(c * _CH, _CH)]],
                buf_v, sem).wait()

            def extract(k, _, c=c):
                j = lax.iota(jnp.int32, _L) + k * _L
                ids16 = ids_v[pl.ds(c * _CH + k * _L, _L)]
                coff = jax.lax.shift_left(
                    jax.lax.bitwise_and(ids16, jnp.int32(1)), jnp.int32(6))
                jo = j + c * _CH

                def inner(f, _2):
                    vals = plsc.load_gather(buf_v, [j, coff + f])
                    plsc.store_scatter(
                        out_v, [jo, jnp.full((_L,), 0, jnp.int32) + f], vals)
                    return _2
                lax.fori_loop(0, D, inner, 0, unroll=4)
                return _
            lax.fori_loop(0, _CH // _L, extract, 0)

        pltpu.sync_copy(out_v, out_hbm.at[pl.ds(base, bpw), :])

    return gather


def kernel(input_ids, labels, alibi, attention_mask, embed_table):
    V, D = embed_table.shape
    ids = input_ids.reshape(-1).astype(jnp.int32)
    B = ids.shape[0]
    table2 = embed_table.reshape(V // 2, 2 * D)
    hidden = _make_gather(B, D)(table2, ids)
    hidden = hidden.reshape(input_ids.shape + (D,))
    return (hidden, labels, alibi, attention_mask)
